# Initial kernel scaffold; baseline (speedup 1.0000x reference)
import jax
import jax.numpy as jnp
from jax.experimental import pallas as pl


def kernel(logits):
    raise NotImplementedError("write your pallas kernel here")


# speedup vs baseline: 176.3064x; 176.3064x over previous
import numpy as np
import jax
import jax.numpy as jnp
from jax import lax
from jax.experimental import pallas as pl

T = 1.3
QHAT = 0.92
LAMDA = 0.15
KREG = 5
TOPK = 12
ROWS = 8

_MSK = np.zeros(TOPK, np.float32)
_MSK[KREG:] = np.float32(LAMDA)
_REGCS = np.cumsum(_MSK).astype(np.float32)


def _body(x_ref, u_ref, mask_ref, sizes_ref):
    x = x_ref[...]
    n = x.shape[1]
    y = x / np.float32(T)
    m = jnp.max(y, axis=1, keepdims=True)
    z = jnp.sum(jnp.exp(y - m), axis=1, keepdims=True)

    iota = lax.broadcasted_iota(jnp.int32, y.shape, 1)
    w = y
    tops = []
    for _ in range(TOPK):
        mk = jnp.max(w, axis=1, keepdims=True)
        ik = jnp.max(jnp.where(w == mk, iota, -1), axis=1, keepdims=True)
        w = jnp.where(iota == ik, -jnp.inf, w)
        tops.append(mk)

    s = [jnp.exp(t - m) / z for t in tops]
    cs = []
    acc = s[0]
    cs.append(acc)
    for k in range(1, TOPK):
        acc = acc + s[k]
        cs.append(acc)
    ord_reg = [s[k] + (np.float32(LAMDA) if k >= KREG else np.float32(0.0))
               for k in range(TOPK)]
    cs_reg = [cs[k] + _REGCS[k] for k in range(TOPK)]

    cnt = jnp.zeros_like(tops[0], dtype=jnp.int32)
    for k in range(TOPK):
        cnt = cnt + (cs_reg[k] <= np.float32(QHAT)).astype(jnp.int32)
    sizes_base = cnt + 1

    idx = sizes_base - 1
    ord_at = jnp.zeros_like(s[0])
    cs_at = jnp.zeros_like(s[0])
    for k in range(TOPK):
        sel = idx == k
        ord_at = jnp.where(sel, ord_reg[k], ord_at)
        cs_at = jnp.where(sel, cs_reg[k], cs_at)
    v = (cs_at - np.float32(QHAT)) / ord_at

    u = u_ref[...].reshape(ROWS, 1)
    sizes = sizes_base - (u <= v).astype(jnp.int32)

    cutoff = jnp.full_like(s[0], jnp.inf)
    for k in range(TOPK):
        cutoff = jnp.where(sizes - 1 == k, tops[k], cutoff)
    mask_ref[...] = y >= cutoff
    sizes_ref[...] = sizes.reshape(1, 1, ROWS)


def kernel(logits):
    b, n = logits.shape
    g = b // ROWS
    u = jax.random.uniform(jax.random.key(1), (b,), dtype=logits.dtype)
    u3 = u.reshape(g, 1, ROWS)

    mask, sizes3 = pl.pallas_call(
        _body,
        grid=(g,),
        in_specs=[
            pl.BlockSpec((ROWS, n), lambda i: (i, 0)),
            pl.BlockSpec((1, 1, ROWS), lambda i: (i, 0, 0)),
        ],
        out_specs=[
            pl.BlockSpec((ROWS, n), lambda i: (i, 0)),
            pl.BlockSpec((1, 1, ROWS), lambda i: (i, 0, 0)),
        ],
        out_shape=[
            jax.ShapeDtypeStruct((b, n), jnp.bool_),
            jax.ShapeDtypeStruct((g, 1, ROWS), jnp.int32),
        ],
    )(logits, u3)

    return (logits, sizes3.reshape(b), mask)
